# baseline (speedup 1.0000x reference)
import functools

import jax
import jax.numpy as jnp
from jax import lax
from jax.experimental import pallas as pl
from jax.experimental.pallas import tpu as pltpu
from jax.experimental.pallas import tpu_sc as plsc

_B = 16384
_D = 64
_NC = 2
_NS = 16
_L = 16
_NW = _NC * _NS
_BPW = _B // _NW
_NB = 128
_NBLK = _BPW // _NB


def _sc_partials(feats2, qidx, hoff, tab2):
    mesh = plsc.VectorSubcoreMesh(core_axis_name="c", subcore_axis_name="s")

    @functools.partial(
        pl.kernel,
        mesh=mesh,
        out_type=jax.ShapeDtypeStruct((_NW, _L), jnp.float32),
        scratch_types=[
            pltpu.VMEM((_BPW,), jnp.int32),
            pltpu.VMEM((_BPW,), jnp.int32),
            pltpu.VMEM((_NB, 2 * _D), jnp.float32),
            pltpu.VMEM((_NB // 2, 2 * _D), jnp.float32),
            pltpu.VMEM((_L,), jnp.float32),
            pltpu.SemaphoreType.DMA,
            pltpu.SemaphoreType.DMA,
        ],
    )
    def k(feat_hbm, q_hbm, h_hbm, tab_hbm, out_hbm, q_v, h_v, rows_v, feat_v,
          acc_v, sem_g, sem_f):
        wid = lax.axis_index("s") * _NC + lax.axis_index("c")
        base = wid * _BPW
        pltpu.sync_copy(q_hbm.at[pl.ds(base, _BPW)], q_v)
        pltpu.sync_copy(h_hbm.at[pl.ds(base, _BPW)], h_v)

        zero = jnp.zeros((_L,), jnp.float32)
        accs = (zero,) * 4
        for b in range(_NBLK):
            gcp = pltpu.async_copy(
                tab_hbm.at[q_v.at[pl.ds(b * _NB, _NB)]], rows_v, sem_g)
            fstart = pl.multiple_of((base + b * _NB) // 2, 64)
            fcp = pltpu.async_copy(
                feat_hbm.at[pl.ds(fstart, _NB // 2)], feat_v, sem_f)
            gcp.wait()
            fcp.wait()

            def body(g, accs, b=b):
                hv = h_v[pl.ds(b * _NB + g * _L, _L)]
                a0, a1, a2, a3 = accs
                for j in range(_L):
                    i = g * _L + j
                    co = hv[j]
                    fr = i // 2
                    fo = (i % 2) * _D
                    d0 = (feat_v[fr, pl.ds(fo, _L)]
                          - rows_v[i, pl.ds(co, _L)])
                    d1 = (feat_v[fr, pl.ds(fo + _L, _L)]
                          - rows_v[i, pl.ds(co + _L, _L)])
                    d2 = (feat_v[fr, pl.ds(fo + 2 * _L, _L)]
                          - rows_v[i, pl.ds(co + 2 * _L, _L)])
                    d3 = (feat_v[fr, pl.ds(fo + 3 * _L, _L)]
                          - rows_v[i, pl.ds(co + 3 * _L, _L)])
                    a0 = a0 + d0 * d0
                    a1 = a1 + d1 * d1
                    a2 = a2 + d2 * d2
                    a3 = a3 + d3 * d3
                return (a0, a1, a2, a3)

            accs = lax.fori_loop(0, _NB // _L, body, accs)

        inv_n = 1.0 / (_B * _D)
        acc_v[...] = (accs[0] + accs[1] + accs[2] + accs[3]) * inv_n
        pltpu.sync_copy(acc_v, out_hbm.at[wid])

    return k(feats2, qidx, hoff, tab2)


def kernel(features, labels, centers):
    labels = labels.astype(jnp.int32)
    qidx = lax.shift_right_logical(labels, 1)
    hoff = lax.shift_left(jnp.bitwise_and(labels, 1), 6)
    tab2 = centers.reshape(50000, 2 * _D)
    feats2 = features.reshape(_B // 2, 2 * _D)
    partials = _sc_partials(feats2, qidx, hoff, tab2)
    return jnp.sum(partials)
